# baseline (speedup 1.0000x reference)
import jax
import jax.numpy as jnp
from jax.experimental import pallas as pl
from jax.experimental.pallas import tpu as pltpu

B, N, F, C = 8, 512, 128, 16
NG = 10
FCN = 64
_WMAP = [0, 1, 2, 3, 4, 5, 6, 7, 6, 7]


def _gcn_body(*refs):
    x_refs = refs[0:NG]
    a_refs = refs[NG:2 * NG]
    w_refs = refs[2 * NG:2 * NG + 8]
    out_ref = refs[-1]
    for i in range(NG):
        xw = jnp.dot(x_refs[i][0], w_refs[_WMAP[i]][...],
                     preferred_element_type=jnp.float32)
        h = jnp.maximum(jnp.dot(a_refs[i][0], xw,
                                preferred_element_type=jnp.float32), 0.0)
        out_ref[0, i] = h.T


def _head_body(xct_ref, wd0_ref, bd0_ref, wd1_ref, bd1_ref, out_ref, acc_ref):
    i = pl.program_id(0)
    lhs3 = xct_ref[:, 0]
    t = jnp.transpose(lhs3, (2, 1, 0))
    f = t.reshape(N * C, B)
    p = jax.lax.dot_general(f, wd0_ref[...], (((0,), (0,)), ((), ())),
                            preferred_element_type=jnp.float32)

    @pl.when(i == 0)
    def _():
        acc_ref[...] = p

    @pl.when(i > 0)
    def _():
        acc_ref[...] = acc_ref[...] + p

    @pl.when(i == NG - 1)
    def _():
        o1 = jnp.maximum(acc_ref[...] + bd0_ref[...], 0.0)
        out_ref[...] = jnp.dot(o1, wd1_ref[...],
                               preferred_element_type=jnp.float32) + bd1_ref[...]


def kernel(x1, a1, x2, a2, x3, a3, x4, a4, x5, a5, x6, a6, x7, a7, x8, a8,
           x9, a9, x10, a10, Wg0, Wg1, Wg2, Wg3, Wg4, Wg5, Wg6, Wg7,
           Wd0, bd0, Wd1, bd1):
    xs = [x1, x2, x3, x4, x5, x6, x7, x8, x9, x10]
    adjs = [a1, a2, a3, a4, a5, a6, a7, a8, a9, a10]
    wgs = [Wg0, Wg1, Wg2, Wg3, Wg4, Wg5, Wg6, Wg7]

    xct = pl.pallas_call(
        _gcn_body,
        grid=(B,),
        in_specs=(
            [pl.BlockSpec((1, N, F), lambda b: (b, 0, 0)) for _ in range(NG)]
            + [pl.BlockSpec((1, N, N), lambda b: (b, 0, 0)) for _ in range(NG)]
            + [pl.BlockSpec((F, C), lambda b: (0, 0)) for _ in range(8)]
        ),
        out_specs=pl.BlockSpec((1, NG, C, N), lambda b: (b, 0, 0, 0)),
        out_shape=jax.ShapeDtypeStruct((B, NG, C, N), jnp.float32),
    )(*xs, *adjs, *wgs)

    o2 = pl.pallas_call(
        _head_body,
        grid=(NG,),
        in_specs=(
            pl.BlockSpec((B, 1, C, N), lambda i: (0, i, 0, 0)),
            pl.BlockSpec((N * C, FCN), lambda i: (i, 0)),
            pl.BlockSpec((1, FCN), lambda i: (0, 0)),
            pl.BlockSpec((FCN, 1), lambda i: (0, 0)),
            pl.BlockSpec((1, 1), lambda i: (0, 0)),
        ),
        out_specs=pl.BlockSpec((B, 1), lambda i: (0, 0)),
        out_shape=jax.ShapeDtypeStruct((B, 1), jnp.float32),
        scratch_shapes=[pltpu.VMEM((B, FCN), jnp.float32)],
    )(xct, Wd0, bd0.reshape(1, FCN), Wd1, bd1.reshape(1, 1))
    return o2


# speedup vs baseline: 1.8563x; 1.8563x over previous
import jax
import jax.numpy as jnp
from jax.experimental import pallas as pl
from jax.experimental.pallas import tpu as pltpu

B, N, F, C = 8, 512, 128, 16
NG = 10
FCN = 64
_WMAP = [0, 1, 2, 3, 4, 5, 6, 7, 6, 7]


def _gcn_body(*refs):
    x_refs = refs[0:NG]
    a_refs = refs[NG:2 * NG]
    w_refs = refs[2 * NG:2 * NG + 8]
    out_ref = refs[-1]
    for i in range(NG):
        xw = jax.lax.dot_general(
            x_refs[i][0], w_refs[_WMAP[i]][...],
            (((1,), (1,)), ((), ())),
            preferred_element_type=jnp.float32)
        h = jnp.maximum(jnp.dot(a_refs[i][0], xw,
                                preferred_element_type=jnp.float32), 0.0)
        out_ref[0, i] = h.T


def _head_body(xct_ref, wdt_ref, bd0_ref, wd1t_ref, bd1_ref, out_ref, acc_ref):
    i = pl.program_id(0)
    lhs3 = xct_ref[:, 0]
    t = jnp.transpose(lhs3, (2, 1, 0))
    f = t.reshape(N * C, B)
    p = jax.lax.dot_general(wdt_ref[...], f, (((1,), (0,)), ((), ())),
                            preferred_element_type=jnp.float32)

    @pl.when(i == 0)
    def _():
        acc_ref[...] = p

    @pl.when(i > 0)
    def _():
        acc_ref[...] = acc_ref[...] + p

    @pl.when(i == NG - 1)
    def _():
        o1t = jnp.maximum(acc_ref[...] + bd0_ref[...], 0.0)
        out_ref[...] = jax.lax.dot_general(
            wd1t_ref[...], o1t, (((1,), (0,)), ((), ())),
            preferred_element_type=jnp.float32) + bd1_ref[...]


def kernel(x1, a1, x2, a2, x3, a3, x4, a4, x5, a5, x6, a6, x7, a7, x8, a8,
           x9, a9, x10, a10, Wg0, Wg1, Wg2, Wg3, Wg4, Wg5, Wg6, Wg7,
           Wd0, bd0, Wd1, bd1):
    xs = [x1, x2, x3, x4, x5, x6, x7, x8, x9, x10]
    adjs = [a1, a2, a3, a4, a5, a6, a7, a8, a9, a10]
    wgts = [W.T for W in (Wg0, Wg1, Wg2, Wg3, Wg4, Wg5, Wg6, Wg7)]

    xct = pl.pallas_call(
        _gcn_body,
        grid=(B,),
        in_specs=(
            [pl.BlockSpec((1, N, F), lambda b: (b, 0, 0)) for _ in range(NG)]
            + [pl.BlockSpec((1, N, N), lambda b: (b, 0, 0)) for _ in range(NG)]
            + [pl.BlockSpec((C, F), lambda b: (0, 0)) for _ in range(8)]
        ),
        out_specs=pl.BlockSpec((1, NG, C, N), lambda b: (b, 0, 0, 0)),
        out_shape=jax.ShapeDtypeStruct((B, NG, C, N), jnp.float32),
    )(*xs, *adjs, *wgts)

    o2t = pl.pallas_call(
        _head_body,
        grid=(NG,),
        in_specs=(
            pl.BlockSpec((B, 1, C, N), lambda i: (0, i, 0, 0)),
            pl.BlockSpec((FCN, N * C), lambda i: (0, i)),
            pl.BlockSpec((FCN, 1), lambda i: (0, 0)),
            pl.BlockSpec((1, FCN), lambda i: (0, 0)),
            pl.BlockSpec((1, 1), lambda i: (0, 0)),
        ),
        out_specs=pl.BlockSpec((1, B), lambda i: (0, 0)),
        out_shape=jax.ShapeDtypeStruct((1, B), jnp.float32),
        scratch_shapes=[pltpu.VMEM((FCN, B), jnp.float32)],
    )(xct, Wd0.T, bd0.reshape(FCN, 1), Wd1.T, bd1.reshape(1, 1))
    return o2t.T
